# baseline (speedup 1.0000x reference)
import jax
import jax.numpy as jnp
from jax.experimental import pallas as pl
from jax.experimental.pallas import tpu as pltpu

_N = 16384
_D = 1024
_CHUNK = 512
_NCHUNK = _N // _CHUNK
_NBUF = 8


def _copy_body(in_hbm, out_hbm, *scratch):
    bufs = scratch[:_NBUF]
    isems = scratch[_NBUF:2 * _NBUF]
    osems = scratch[2 * _NBUF:]

    def in_cp(i, b):
        return pltpu.make_async_copy(
            in_hbm.at[pl.ds(i * _CHUNK, _CHUNK), :], bufs[b], isems[b])

    def out_cp(i, b):
        return pltpu.make_async_copy(
            bufs[b], out_hbm.at[pl.ds(i * _CHUNK, _CHUNK), :], osems[b])

    for b in range(_NBUF):
        in_cp(b, b).start()
    for i in range(_NCHUNK):
        b = i % _NBUF
        if i >= _NBUF:
            out_cp(i - _NBUF, b).wait()
            in_cp(i, b).start()
        in_cp(i, b).wait()
        out_cp(i, b).start()
    for i in range(_NCHUNK - _NBUF, _NCHUNK):
        out_cp(i, i % _NBUF).wait()


def kernel(input):
    return pl.pallas_call(
        _copy_body,
        in_specs=[pl.BlockSpec(memory_space=pltpu.MemorySpace.HBM)],
        out_specs=pl.BlockSpec(memory_space=pltpu.MemorySpace.HBM),
        out_shape=jax.ShapeDtypeStruct((_N, _D), jnp.float32),
        scratch_shapes=(
            [pltpu.VMEM((_CHUNK, _D), jnp.float32) for _ in range(_NBUF)]
            + [pltpu.SemaphoreType.DMA for _ in range(2 * _NBUF)]
        ),
    )(input)


# speedup vs baseline: 1.4536x; 1.4536x over previous
import jax
import jax.numpy as jnp
from jax.experimental import pallas as pl
from jax.experimental.pallas import tpu as pltpu

_N = 16384
_D = 1024
_CHUNK = 2048
_NCHUNK = _N // _CHUNK
_NBUF = 3


def _copy_body(in_hbm, out_hbm, *scratch):
    bufs = scratch[:_NBUF]
    isems = scratch[_NBUF:2 * _NBUF]
    osems = scratch[2 * _NBUF:]

    def in_cp(i, b):
        return pltpu.make_async_copy(
            in_hbm.at[pl.ds(i * _CHUNK, _CHUNK), :], bufs[b], isems[b])

    def out_cp(i, b):
        return pltpu.make_async_copy(
            bufs[b], out_hbm.at[pl.ds(i * _CHUNK, _CHUNK), :], osems[b])

    for b in range(_NBUF):
        in_cp(b, b).start()
    for i in range(_NCHUNK):
        b = i % _NBUF
        if i >= _NBUF:
            out_cp(i - _NBUF, b).wait()
            in_cp(i, b).start()
        in_cp(i, b).wait()
        out_cp(i, b).start()
    for i in range(_NCHUNK - _NBUF, _NCHUNK):
        out_cp(i, i % _NBUF).wait()


def kernel(input):
    return pl.pallas_call(
        _copy_body,
        in_specs=[pl.BlockSpec(memory_space=pltpu.MemorySpace.HBM)],
        out_specs=pl.BlockSpec(memory_space=pltpu.MemorySpace.HBM),
        out_shape=jax.ShapeDtypeStruct((_N, _D), jnp.float32),
        scratch_shapes=(
            [pltpu.VMEM((_CHUNK, _D), jnp.float32) for _ in range(_NBUF)]
            + [pltpu.SemaphoreType.DMA for _ in range(2 * _NBUF)]
        ),
    )(input)


# speedup vs baseline: 1.7158x; 1.1804x over previous
import jax
import jax.numpy as jnp
from jax.experimental import pallas as pl
from jax.experimental.pallas import tpu as pltpu

_N = 16384
_D = 1024
_BLOCK_ROWS = 3584


def _copy_body(x_ref, o_ref):
    o_ref[...] = x_ref[...]


def kernel(input):
    return pl.pallas_call(
        _copy_body,
        grid=(pl.cdiv(_N, _BLOCK_ROWS),),
        in_specs=[pl.BlockSpec((_BLOCK_ROWS, _D), lambda i: (i, 0))],
        out_specs=pl.BlockSpec((_BLOCK_ROWS, _D), lambda i: (i, 0)),
        out_shape=jax.ShapeDtypeStruct((_N, _D), jnp.float32),
        compiler_params=pltpu.CompilerParams(
            dimension_semantics=("arbitrary",),
        ),
    )(input)
